# baseline (speedup 1.0000x reference)
import functools

import jax
import jax.numpy as jnp
from jax import lax
from jax.experimental import pallas as pl
from jax.experimental.pallas import tpu as pltpu
from jax.experimental.pallas import tpu_sc as plsc

HID = 1024
EPS = 1e-12
N_CHUNKS = 2


def _sc_gather(word_table, flat_ids):
    n = flat_ids.shape[0]
    info = plsc.get_sparse_core_info()
    nw = info.num_cores * info.num_subcores
    per_w = n // nw
    chunk = 64
    n_chunks = per_w // chunk
    mesh = plsc.VectorSubcoreMesh(core_axis_name="c", subcore_axis_name="s")

    @functools.partial(
        pl.kernel,
        mesh=mesh,
        out_type=jax.ShapeDtypeStruct((n, HID), jnp.float32),
        scratch_types=[
            pltpu.VMEM((chunk,), jnp.int32),
            pltpu.VMEM((chunk, HID), jnp.float32),
            pltpu.SemaphoreType.DMA,
        ],
    )
    def gather_kernel(ids_hbm, table_hbm, out_hbm, idx_v, rows_v, sem):
        wid = lax.axis_index("s") * info.num_cores + lax.axis_index("c")
        base = wid * per_w

        def body(c, carry):
            tok = base + c * chunk
            pltpu.sync_copy(ids_hbm.at[pl.ds(tok, chunk)], idx_v)
            pltpu.async_copy(table_hbm.at[idx_v], rows_v, sem).wait()
            pltpu.sync_copy(rows_v, out_hbm.at[pl.ds(tok, chunk)])
            return carry

        lax.fori_loop(0, n_chunks, body, 0)

    return gather_kernel(flat_ids, word_table)


def _tc_add_ln_chunk(gathered_c, pos_s, const_row, dtf, didf, tf_c, idf_c,
                     gamma, beta, out_buf, chunk_idx):
    bk, s, _ = gathered_c.shape
    blk = 256
    s_blocks = s // blk

    def body(g_ref, p_ref, c_ref, dtf_ref, didf_ref, tf_ref, idf_ref,
             gam_ref, bet_ref, buf_ref, o_ref):
        x = g_ref[...] + p_ref[...][None] + c_ref[...][None]
        x = x + tf_ref[...] * dtf_ref[...][None]
        x = x + idf_ref[...] * didf_ref[...][None]
        mu = jnp.mean(x, axis=-1, keepdims=True)
        xc = x - mu
        var = jnp.mean(xc * xc, axis=-1, keepdims=True)
        y = xc * lax.rsqrt(var + EPS)
        o_ref[...] = y * gam_ref[...][None] + bet_ref[...][None]

    row_spec = pl.BlockSpec((1, HID), lambda i: (0, 0))
    return pl.pallas_call(
        body,
        grid=(s_blocks,),
        in_specs=[
            pl.BlockSpec((bk, blk, HID), lambda i: (0, i, 0)),
            pl.BlockSpec((blk, HID), lambda i: (i, 0)),
            row_spec,
            row_spec,
            row_spec,
            pl.BlockSpec((bk, blk, 1), lambda i: (0, i, 0)),
            pl.BlockSpec((bk, blk, 1), lambda i: (0, i, 0)),
            row_spec,
            row_spec,
            pl.BlockSpec(memory_space=pl.ANY),
        ],
        out_specs=pl.BlockSpec((bk, blk, HID), lambda i: (chunk_idx, i, 0)),
        out_shape=jax.ShapeDtypeStruct(out_buf.shape, jnp.float32),
        input_output_aliases={9: 0},
    )(gathered_c, pos_s, const_row, dtf, didf, tf_c, idf_c, gamma, beta,
      out_buf)


def kernel(input_ids, tf_type, idf_type, word_table, pos_table, type_table,
           tf_table, idf_table, gamma, beta):
    b, s = input_ids.shape
    ids = input_ids.astype(jnp.int32)

    pos_s = pos_table[:s]
    const_row = (type_table[0] + tf_table[0] + idf_table[0])[None, :]
    dtf = (tf_table[1] - tf_table[0])[None, :]
    didf = (idf_table[1] - idf_table[0])[None, :]
    tf_f = tf_type.reshape(b, s, 1).astype(jnp.float32)
    idf_f = idf_type.reshape(b, s, 1).astype(jnp.float32)
    gam = gamma[None, :]
    bet = beta[None, :]

    bk = b // N_CHUNKS
    gathered = [
        _sc_gather(word_table, ids[c * bk:(c + 1) * bk].reshape(-1))
        .reshape(bk, s, HID)
        for c in range(N_CHUNKS)
    ]

    out = jnp.zeros((b, s, HID), jnp.float32)
    for c in range(N_CHUNKS):
        out = _tc_add_ln_chunk(
            gathered[c], pos_s, const_row, dtf, didf,
            tf_f[c * bk:(c + 1) * bk], idf_f[c * bk:(c + 1) * bk],
            gam, bet, out, c)
    return out


# speedup vs baseline: 1.2307x; 1.2307x over previous
import functools

import jax
import jax.numpy as jnp
from jax import lax
from jax.experimental import pallas as pl
from jax.experimental.pallas import tpu as pltpu
from jax.experimental.pallas import tpu_sc as plsc

HID = 1024
EPS = 1e-12


def _sc_gather(word_table, flat_ids):
    n = flat_ids.shape[0]
    info = plsc.get_sparse_core_info()
    nw = info.num_cores * info.num_subcores
    per_w = n // nw
    chunk = 32
    n_chunks = per_w // chunk
    mesh = plsc.VectorSubcoreMesh(core_axis_name="c", subcore_axis_name="s")

    @functools.partial(
        pl.kernel,
        mesh=mesh,
        out_type=jax.ShapeDtypeStruct((n, HID), jnp.float32),
        scratch_types=[
            pltpu.VMEM((per_w,), jnp.int32),
            pltpu.VMEM((2, chunk, HID), jnp.float32),
            pltpu.SemaphoreType.DMA,
            pltpu.SemaphoreType.DMA,
        ],
    )
    def gather_kernel(ids_hbm, table_hbm, out_hbm, idx_v, rows_v, gsem, wsem):
        wid = lax.axis_index("s") * info.num_cores + lax.axis_index("c")
        base = wid * per_w
        pltpu.sync_copy(ids_hbm.at[pl.ds(base, per_w)], idx_v)

        gathers = [
            pltpu.make_async_copy(
                table_hbm.at[idx_v.at[pl.ds(c * chunk, chunk)]],
                rows_v.at[c % 2], gsem)
            for c in range(n_chunks)
        ]
        writes = [
            pltpu.make_async_copy(
                rows_v.at[c % 2], out_hbm.at[pl.ds(base + c * chunk, chunk)],
                wsem)
            for c in range(n_chunks)
        ]

        gathers[0].start()
        for c in range(n_chunks):
            if c + 1 < n_chunks:
                if c >= 1:
                    writes[c - 1].wait()
                gathers[c + 1].start()
            gathers[c].wait()
            writes[c].start()
        if n_chunks >= 2:
            writes[n_chunks - 2].wait()
        writes[n_chunks - 1].wait()

    return gather_kernel(flat_ids, word_table)


def _tc_add_ln(gathered, pos_s, const_row, dtf, didf, tf_f, idf_f, gamma, beta):
    b, s, _ = gathered.shape
    blk = 256
    s_blocks = s // blk

    def body(g_ref, p_ref, c_ref, dtf_ref, didf_ref, tf_ref, idf_ref,
             gam_ref, bet_ref, o_ref):
        x = g_ref[...] + p_ref[...][None] + c_ref[...][None]
        x = x + tf_ref[...] * dtf_ref[...][None]
        x = x + idf_ref[...] * didf_ref[...][None]
        mu = jnp.mean(x, axis=-1, keepdims=True)
        s2 = jnp.mean(x * x, axis=-1, keepdims=True)
        var = s2 - mu * mu
        y = (x - mu) * lax.rsqrt(var + EPS)
        o_ref[...] = y * gam_ref[...][None] + bet_ref[...][None]

    row_spec = pl.BlockSpec((1, HID), lambda i: (0, 0))
    return pl.pallas_call(
        body,
        grid=(s_blocks,),
        in_specs=[
            pl.BlockSpec((b, blk, HID), lambda i: (0, i, 0)),
            pl.BlockSpec((blk, HID), lambda i: (i, 0)),
            row_spec,
            row_spec,
            row_spec,
            pl.BlockSpec((b, blk, 1), lambda i: (0, i, 0)),
            pl.BlockSpec((b, blk, 1), lambda i: (0, i, 0)),
            row_spec,
            row_spec,
        ],
        out_specs=pl.BlockSpec((b, blk, HID), lambda i: (0, i, 0)),
        out_shape=jax.ShapeDtypeStruct((b, s, HID), jnp.float32),
    )(gathered, pos_s, const_row, dtf, didf, tf_f, idf_f, gamma, beta)


def kernel(input_ids, tf_type, idf_type, word_table, pos_table, type_table,
           tf_table, idf_table, gamma, beta):
    b, s = input_ids.shape
    flat_ids = input_ids.reshape(-1).astype(jnp.int32)

    gathered = _sc_gather(word_table, flat_ids).reshape(b, s, HID)

    pos_s = pos_table[:s]
    const_row = (type_table[0] + tf_table[0] + idf_table[0])[None, :]
    dtf = (tf_table[1] - tf_table[0])[None, :]
    didf = (idf_table[1] - idf_table[0])[None, :]
    tf_f = tf_type.reshape(b, s, 1).astype(jnp.float32)
    idf_f = idf_type.reshape(b, s, 1).astype(jnp.float32)

    return _tc_add_ln(gathered, pos_s, const_row, dtf, didf, tf_f, idf_f,
                      gamma[None, :], beta[None, :])


# speedup vs baseline: 1.2446x; 1.0114x over previous
import functools

import jax
import jax.numpy as jnp
from jax import lax
from jax.experimental import pallas as pl
from jax.experimental.pallas import tpu as pltpu
from jax.experimental.pallas import tpu_sc as plsc

HID = 1024
EPS = 1e-12


def _sc_gather(word_table, flat_ids):
    n = flat_ids.shape[0]
    info = plsc.get_sparse_core_info()
    nw = info.num_cores * info.num_subcores
    per_w = n // nw
    chunk = 32
    n_chunks = per_w // chunk
    mesh = plsc.VectorSubcoreMesh(core_axis_name="c", subcore_axis_name="s")

    @functools.partial(
        pl.kernel,
        mesh=mesh,
        out_type=jax.ShapeDtypeStruct((n, HID), jnp.float32),
        scratch_types=[
            pltpu.VMEM((per_w,), jnp.int32),
            pltpu.VMEM((2, chunk, HID), jnp.float32),
            pltpu.SemaphoreType.DMA,
            pltpu.SemaphoreType.DMA,
        ],
    )
    def gather_kernel(ids_hbm, table_hbm, out_hbm, idx_v, rows_v, gsem, wsem):
        wid = lax.axis_index("s") * info.num_cores + lax.axis_index("c")
        base = wid * per_w
        pltpu.sync_copy(ids_hbm.at[pl.ds(base, per_w)], idx_v)

        gathers = [
            pltpu.make_async_copy(
                table_hbm.at[idx_v.at[pl.ds(c * chunk, chunk)]],
                rows_v.at[c % 2], gsem)
            for c in range(n_chunks)
        ]
        writes = [
            pltpu.make_async_copy(
                rows_v.at[c % 2], out_hbm.at[pl.ds(base + c * chunk, chunk)],
                wsem)
            for c in range(n_chunks)
        ]

        gathers[0].start()
        for c in range(n_chunks):
            if c + 1 < n_chunks:
                if c >= 1:
                    writes[c - 1].wait()
                gathers[c + 1].start()
            gathers[c].wait()
            writes[c].start()
        if n_chunks >= 2:
            writes[n_chunks - 2].wait()
        writes[n_chunks - 1].wait()

    return gather_kernel(flat_ids, word_table)


def _tc_add_ln(gathered, pos_s, const_row, dtf, didf, tf_f, idf_f, gamma, beta):
    b, s, _ = gathered.shape
    blk = 512
    s_blocks = s // blk

    def body(g_ref, p_ref, c_ref, dtf_ref, didf_ref, tf_ref, idf_ref,
             gam_ref, bet_ref, o_ref):
        x = g_ref[...] + p_ref[...][None] + c_ref[...][None]
        x = x + tf_ref[...] * dtf_ref[...][None]
        x = x + idf_ref[...] * didf_ref[...][None]
        mu = jnp.mean(x, axis=-1, keepdims=True)
        s2 = jnp.mean(x * x, axis=-1, keepdims=True)
        var = s2 - mu * mu
        y = (x - mu) * lax.rsqrt(var + EPS)
        o_ref[...] = y * gam_ref[...][None] + bet_ref[...][None]

    row_spec = pl.BlockSpec((1, HID), lambda i: (0, 0))
    return pl.pallas_call(
        body,
        grid=(s_blocks,),
        in_specs=[
            pl.BlockSpec((b, blk, HID), lambda i: (0, i, 0)),
            pl.BlockSpec((blk, HID), lambda i: (i, 0)),
            row_spec,
            row_spec,
            row_spec,
            pl.BlockSpec((b, blk, 1), lambda i: (0, i, 0)),
            pl.BlockSpec((b, blk, 1), lambda i: (0, i, 0)),
            row_spec,
            row_spec,
        ],
        out_specs=pl.BlockSpec((b, blk, HID), lambda i: (0, i, 0)),
        out_shape=jax.ShapeDtypeStruct((b, s, HID), jnp.float32),
    )(gathered, pos_s, const_row, dtf, didf, tf_f, idf_f, gamma, beta)


def kernel(input_ids, tf_type, idf_type, word_table, pos_table, type_table,
           tf_table, idf_table, gamma, beta):
    b, s = input_ids.shape
    flat_ids = input_ids.reshape(-1).astype(jnp.int32)

    gathered = _sc_gather(word_table, flat_ids).reshape(b, s, HID)

    pos_s = pos_table[:s]
    const_row = (type_table[0] + tf_table[0] + idf_table[0])[None, :]
    dtf = (tf_table[1] - tf_table[0])[None, :]
    didf = (idf_table[1] - idf_table[0])[None, :]
    tf_f = tf_type.reshape(b, s, 1).astype(jnp.float32)
    idf_f = idf_type.reshape(b, s, 1).astype(jnp.float32)

    return _tc_add_ln(gathered, pos_s, const_row, dtf, didf, tf_f, idf_f,
                      gamma[None, :], beta[None, :])


# speedup vs baseline: 1.3369x; 1.0741x over previous
import functools

import jax
import jax.numpy as jnp
from jax import lax
from jax.experimental import pallas as pl
from jax.experimental.pallas import tpu as pltpu
from jax.experimental.pallas import tpu_sc as plsc

HID = 1024
EPS = 1e-12


def _sc_gather(word_table, flat_ids):
    n = flat_ids.shape[0]
    info = plsc.get_sparse_core_info()
    nw = info.num_cores * info.num_subcores
    per_w = n // nw
    chunk = 32
    n_chunks = per_w // chunk
    mesh = plsc.VectorSubcoreMesh(core_axis_name="c", subcore_axis_name="s")

    @functools.partial(
        pl.kernel,
        mesh=mesh,
        out_type=jax.ShapeDtypeStruct((n, HID), jnp.float32),
        scratch_types=[
            pltpu.VMEM((per_w,), jnp.int32),
            pltpu.VMEM((2, chunk, HID), jnp.float32),
            pltpu.SemaphoreType.DMA,
            pltpu.SemaphoreType.DMA,
        ],
    )
    def gather_kernel(ids_hbm, table_hbm, out_hbm, idx_v, rows_v, gsem, wsem):
        wid = lax.axis_index("s") * info.num_cores + lax.axis_index("c")
        base = wid * per_w
        pltpu.sync_copy(ids_hbm.at[pl.ds(base, per_w)], idx_v)

        gathers = [
            pltpu.make_async_copy(
                table_hbm.at[idx_v.at[pl.ds(c * chunk, chunk)]],
                rows_v.at[c % 2], gsem)
            for c in range(n_chunks)
        ]
        writes = [
            pltpu.make_async_copy(
                rows_v.at[c % 2], out_hbm.at[pl.ds(base + c * chunk, chunk)],
                wsem)
            for c in range(n_chunks)
        ]

        gathers[0].start()
        for c in range(n_chunks):
            if c + 1 < n_chunks:
                if c >= 1:
                    writes[c - 1].wait()
                gathers[c + 1].start()
            gathers[c].wait()
            writes[c].start()
        if n_chunks >= 2:
            writes[n_chunks - 2].wait()
        writes[n_chunks - 1].wait()

    return gather_kernel(flat_ids, word_table)


def _tc_add_ln(gathered, pos_s, type_table, tf_table, idf_table, tf_type,
               idf_type, gamma, beta):
    b, s, _ = gathered.shape
    blk = 512
    s_blocks = s // blk

    def body(g_ref, p_ref, tt_ref, tft_ref, idft_ref, tf_ref, idf_ref,
             gam_ref, bet_ref, o_ref):
        const_row = tt_ref[0] + tft_ref[0] + idft_ref[0]
        dtf = tft_ref[1] - tft_ref[0]
        didf = idft_ref[1] - idft_ref[0]
        tf_w = tf_ref[...].astype(jnp.float32)[:, :, None]
        idf_w = idf_ref[...].astype(jnp.float32)[:, :, None]
        x = g_ref[...] + p_ref[...][None] + const_row[None, None]
        x = x + tf_w * dtf[None, None]
        x = x + idf_w * didf[None, None]
        mu = jnp.mean(x, axis=-1, keepdims=True)
        s2 = jnp.mean(x * x, axis=-1, keepdims=True)
        var = s2 - mu * mu
        y = (x - mu) * lax.rsqrt(var + EPS)
        o_ref[...] = y * gam_ref[...][None] + bet_ref[...][None]

    row_spec = pl.BlockSpec((1, HID), lambda i: (0, 0))
    table_spec = pl.BlockSpec((2, HID), lambda i: (0, 0))
    flag_spec = pl.BlockSpec((b, blk), lambda i: (0, i))
    return pl.pallas_call(
        body,
        grid=(s_blocks,),
        in_specs=[
            pl.BlockSpec((b, blk, HID), lambda i: (0, i, 0)),
            pl.BlockSpec((blk, HID), lambda i: (i, 0)),
            table_spec,
            table_spec,
            table_spec,
            flag_spec,
            flag_spec,
            row_spec,
            row_spec,
        ],
        out_specs=pl.BlockSpec((b, blk, HID), lambda i: (0, i, 0)),
        out_shape=jax.ShapeDtypeStruct((b, s, HID), jnp.float32),
    )(gathered, pos_s, type_table, tf_table, idf_table, tf_type, idf_type,
      gamma, beta)


def kernel(input_ids, tf_type, idf_type, word_table, pos_table, type_table,
           tf_table, idf_table, gamma, beta):
    b, s = input_ids.shape
    flat_ids = input_ids.reshape(-1).astype(jnp.int32)

    gathered = _sc_gather(word_table, flat_ids).reshape(b, s, HID)

    return _tc_add_ln(gathered, pos_table[:s], type_table, tf_table,
                      idf_table, tf_type, idf_type, gamma[None, :],
                      beta[None, :])


# speedup vs baseline: 1.3973x; 1.0451x over previous
import functools

import jax
import jax.numpy as jnp
from jax import lax
from jax.experimental import pallas as pl
from jax.experimental.pallas import tpu as pltpu
from jax.experimental.pallas import tpu_sc as plsc

HID = 1024
EPS = 1e-12


def _sc_gather(word_table, flat_ids):
    n = flat_ids.shape[0]
    info = plsc.get_sparse_core_info()
    nw = info.num_cores * info.num_subcores
    per_w = n // nw
    chunk = 32
    n_chunks = per_w // chunk
    mesh = plsc.VectorSubcoreMesh(core_axis_name="c", subcore_axis_name="s")

    @functools.partial(
        pl.kernel,
        mesh=mesh,
        out_type=jax.ShapeDtypeStruct((n, HID), jnp.float32),
        scratch_types=[
            pltpu.VMEM((per_w,), jnp.int32),
            pltpu.VMEM((2, chunk, HID), jnp.float32),
            pltpu.SemaphoreType.DMA,
            pltpu.SemaphoreType.DMA,
        ],
    )
    def gather_kernel(ids_hbm, table_hbm, out_hbm, idx_v, rows_v, gsem, wsem):
        wid = lax.axis_index("s") * info.num_cores + lax.axis_index("c")
        base = wid * per_w
        pltpu.sync_copy(ids_hbm.at[pl.ds(base, per_w)], idx_v)

        gathers = [
            pltpu.make_async_copy(
                table_hbm.at[idx_v.at[pl.ds(c * chunk, chunk)]],
                rows_v.at[c % 2], gsem)
            for c in range(n_chunks)
        ]
        writes = [
            pltpu.make_async_copy(
                rows_v.at[c % 2], out_hbm.at[pl.ds(base + c * chunk, chunk)],
                wsem)
            for c in range(n_chunks)
        ]

        gathers[0].start()
        for c in range(n_chunks):
            if c + 1 < n_chunks:
                if c >= 1:
                    writes[c - 1].wait()
                gathers[c + 1].start()
            gathers[c].wait()
            writes[c].start()
        if n_chunks >= 2:
            writes[n_chunks - 2].wait()
        writes[n_chunks - 1].wait()

    return gather_kernel(flat_ids, word_table)


def _tc_add_ln(gathered, pos_s, type_table, tf_table, idf_table, tf_type,
               idf_type, gamma, beta):
    b, s, _ = gathered.shape
    blk = 512
    s_blocks = s // blk

    def body(g_ref, p_ref, tt_ref, tft_ref, idft_ref, tf_ref, idf_ref,
             gam_ref, bet_ref, o_ref):
        const_row = tt_ref[0] + tft_ref[0] + idft_ref[0]
        dtf = tft_ref[1] - tft_ref[0]
        didf = idft_ref[1] - idft_ref[0]
        tf_w = tf_ref[...].astype(jnp.float32)[:, :, None]
        idf_w = idf_ref[...].astype(jnp.float32)[:, :, None]
        x = g_ref[...] + p_ref[...][None] + const_row[None, None]
        x = x + tf_w * dtf[None, None]
        x = x + idf_w * didf[None, None]
        mu = jnp.mean(x, axis=-1, keepdims=True)
        s2 = jnp.mean(x * x, axis=-1, keepdims=True)
        var = s2 - mu * mu
        y = (x - mu) * lax.rsqrt(var + EPS)
        o_ref[...] = y * gam_ref[...][None] + bet_ref[...][None]

    row_spec = pl.BlockSpec((1, HID), lambda i: (0, 0))
    table_spec = pl.BlockSpec((2, HID), lambda i: (0, 0))
    flag_spec = pl.BlockSpec((b, blk), lambda i: (0, i))
    return pl.pallas_call(
        body,
        grid=(s_blocks,),
        in_specs=[
            pl.BlockSpec((b, blk, HID), lambda i: (0, i, 0)),
            pl.BlockSpec((blk, HID), lambda i: (i, 0)),

            table_spec,
            table_spec,
            table_spec,
            flag_spec,
            flag_spec,
            row_spec,
            row_spec,
        ],
        out_specs=pl.BlockSpec((b, blk, HID), lambda i: (0, i, 0)),
        out_shape=jax.ShapeDtypeStruct((b, s, HID), jnp.float32),
    )(gathered, pos_s, type_table, tf_table, idf_table, tf_type, idf_type,
      gamma, beta)


def kernel(input_ids, tf_type, idf_type, word_table, pos_table, type_table,
           tf_table, idf_table, gamma, beta):
    b, s = input_ids.shape
    flat_ids = input_ids.reshape(-1).astype(jnp.int32)

    gathered = _sc_gather(word_table, flat_ids).reshape(b, s, HID)

    return _tc_add_ln(gathered, pos_table, type_table, tf_table,
                      idf_table, tf_type, idf_type, gamma[None, :],
                      beta[None, :])


# speedup vs baseline: 1.4198x; 1.0161x over previous
import functools

import jax
import jax.numpy as jnp
from jax import lax
from jax.experimental import pallas as pl
from jax.experimental.pallas import tpu as pltpu
from jax.experimental.pallas import tpu_sc as plsc

HID = 1024
EPS = 1e-12


def _sc_gather(word_table, flat_ids):
    n = flat_ids.shape[0]
    info = plsc.get_sparse_core_info()
    nw = info.num_cores * info.num_subcores
    per_w = n // nw
    chunk = 32
    n_chunks = per_w // chunk
    mesh = plsc.VectorSubcoreMesh(core_axis_name="c", subcore_axis_name="s")

    @functools.partial(
        pl.kernel,
        mesh=mesh,
        out_type=jax.ShapeDtypeStruct((n, HID), jnp.float32),
        scratch_types=[
            pltpu.VMEM((per_w,), jnp.int32),
            pltpu.VMEM((2, chunk, HID), jnp.float32),
            pltpu.SemaphoreType.DMA,
            pltpu.SemaphoreType.DMA,
        ],
    )
    def gather_kernel(ids_hbm, table_hbm, out_hbm, idx_v, rows_v, gsem, wsem):
        wid = lax.axis_index("s") * info.num_cores + lax.axis_index("c")
        base = wid * per_w
        pltpu.sync_copy(ids_hbm.at[pl.ds(base, per_w)], idx_v)

        def gather_c(c, slot):
            return pltpu.make_async_copy(
                table_hbm.at[idx_v.at[pl.ds(c * chunk, chunk)]],
                rows_v.at[slot], gsem)

        def write_c(c, slot):
            return pltpu.make_async_copy(
                rows_v.at[slot], out_hbm.at[pl.ds(base + c * chunk, chunk)],
                wsem)

        gather_c(0, 0).start()

        def body(c, carry):
            slot = lax.rem(c, 2)
            oslot = 1 - slot

            @pl.when(c + 1 < n_chunks)
            def _():
                @pl.when(c >= 1)
                def _():
                    write_c(0, 0).wait()

                gather_c(c + 1, oslot).start()

            gather_c(c, slot).wait()
            write_c(c, slot).start()
            return carry

        lax.fori_loop(0, n_chunks, body, 0)
        write_c(0, 0).wait()
        write_c(0, 0).wait()

    return gather_kernel(flat_ids, word_table)


def _tc_add_ln(gathered, pos_s, type_table, tf_table, idf_table, tf_type,
               idf_type, gamma, beta):
    b, s, _ = gathered.shape
    blk = 512
    s_blocks = s // blk

    def body(g_ref, p_ref, tt_ref, tft_ref, idft_ref, tf_ref, idf_ref,
             gam_ref, bet_ref, o_ref):
        const_row = tt_ref[0] + tft_ref[0] + idft_ref[0]
        dtf = tft_ref[1] - tft_ref[0]
        didf = idft_ref[1] - idft_ref[0]
        tf_w = tf_ref[...].astype(jnp.float32)[:, :, None]
        idf_w = idf_ref[...].astype(jnp.float32)[:, :, None]
        x = g_ref[...] + p_ref[...][None] + const_row[None, None]
        x = x + tf_w * dtf[None, None]
        x = x + idf_w * didf[None, None]
        mu = jnp.mean(x, axis=-1, keepdims=True)
        s2 = jnp.mean(x * x, axis=-1, keepdims=True)
        var = s2 - mu * mu
        y = (x - mu) * lax.rsqrt(var + EPS)
        o_ref[...] = y * gam_ref[...][None] + bet_ref[...][None]

    row_spec = pl.BlockSpec((1, HID), lambda i: (0, 0))
    table_spec = pl.BlockSpec((2, HID), lambda i: (0, 0))
    flag_spec = pl.BlockSpec((b, blk), lambda i: (0, i))
    return pl.pallas_call(
        body,
        grid=(s_blocks,),
        in_specs=[
            pl.BlockSpec((b, blk, HID), lambda i: (0, i, 0)),
            pl.BlockSpec((blk, HID), lambda i: (i, 0)),

            table_spec,
            table_spec,
            table_spec,
            flag_spec,
            flag_spec,
            row_spec,
            row_spec,
        ],
        out_specs=pl.BlockSpec((b, blk, HID), lambda i: (0, i, 0)),
        out_shape=jax.ShapeDtypeStruct((b, s, HID), jnp.float32),
    )(gathered, pos_s, type_table, tf_table, idf_table, tf_type, idf_type,
      gamma, beta)


def kernel(input_ids, tf_type, idf_type, word_table, pos_table, type_table,
           tf_table, idf_table, gamma, beta):
    b, s = input_ids.shape
    flat_ids = input_ids.reshape(-1).astype(jnp.int32)

    gathered = _sc_gather(word_table, flat_ids).reshape(b, s, HID)

    return _tc_add_ln(gathered, pos_table, type_table, tf_table,
                      idf_table, tf_type, idf_type, gamma[None, :],
                      beta[None, :])
